# baseline (device time: 9464 ns/iter reference)
import jax
import jax.numpy as jnp
from jax import lax
from jax.experimental import pallas as pl
from jax.experimental.pallas import tpu as pltpu

N_GLOBAL = 512
EPS = 1e-5


def kernel(x, gamma, beta):
    m, n = x.shape

    def body(x_ref, gamma_ref, beta_ref, out_ref,
             local_stats, remote_stats, send_sem, recv_sem):
        my_x = lax.axis_index("x")
        my_y = lax.axis_index("y")
        partner = (my_x, 1 - my_y)

        barrier = pltpu.get_barrier_semaphore()
        pl.semaphore_signal(
            barrier, inc=1,
            device_id=partner, device_id_type=pl.DeviceIdType.MESH,
        )
        pl.semaphore_wait(barrier, 1)

        xv = x_ref[:, :].astype(jnp.float32)
        local_stats[:, 0:1] = jnp.sum(xv, axis=1, keepdims=True)
        local_stats[:, 1:2] = jnp.sum(xv * xv, axis=1, keepdims=True)

        rdma = pltpu.make_async_remote_copy(
            src_ref=local_stats,
            dst_ref=remote_stats,
            send_sem=send_sem,
            recv_sem=recv_sem,
            device_id=partner,
            device_id_type=pl.DeviceIdType.MESH,
        )
        rdma.start()
        rdma.wait()

        total = local_stats[:, 0:1] + remote_stats[:, 0:1]
        total_sq = local_stats[:, 1:2] + remote_stats[:, 1:2]
        mean = total / N_GLOBAL
        var = total_sq / N_GLOBAL - mean * mean
        inv = lax.rsqrt(var + EPS)
        g = gamma_ref[0:1, :].astype(jnp.float32)
        b = beta_ref[0:1, :].astype(jnp.float32)
        out_ref[:, :] = (g * ((xv - mean) * inv) + b).astype(out_ref.dtype)

    out_shape = jax.ShapeDtypeStruct((m, n), x.dtype)
    return pl.pallas_call(
        body,
        out_shape=out_shape,
        in_specs=[
            pl.BlockSpec(memory_space=pltpu.VMEM),
            pl.BlockSpec(memory_space=pltpu.VMEM),
            pl.BlockSpec(memory_space=pltpu.VMEM),
        ],
        out_specs=pl.BlockSpec(memory_space=pltpu.VMEM),
        scratch_shapes=[
            pltpu.VMEM((m, 2), jnp.float32),
            pltpu.VMEM((m, 2), jnp.float32),
            pltpu.SemaphoreType.DMA,
            pltpu.SemaphoreType.DMA,
        ],
        compiler_params=pltpu.CompilerParams(collective_id=0),
    )(x, gamma.reshape(1, n), beta.reshape(1, n))
